# baseline (speedup 1.0000x reference)
import functools

import jax
import jax.numpy as jnp
from jax.experimental import pallas as pl
from jax.experimental.pallas import tpu as pltpu

N = 10000
K = 40


def _leaky(v):
    return jnp.where(v >= 0, v, 0.5 * v)


def _encoder_body(x_ref, w1, b1, w2, b2, w3, b3, ws, bs, wh, bh,
                  x1_ref, s_ref, hp_ref):
    x = x_ref[...]
    h = _leaky(jnp.dot(x, w1[...], preferred_element_type=jnp.float32) + b1[...])
    h = _leaky(jnp.dot(h, w2[...], preferred_element_type=jnp.float32) + b2[...])
    x1 = _leaky(jnp.dot(h, w3[...], preferred_element_type=jnp.float32) + b3[...])
    x1_ref[...] = x1
    s_ref[...] = jnp.dot(x1, ws[...], preferred_element_type=jnp.float32) + bs[...]
    hp_ref[...] = jnp.dot(x1, wh[...], preferred_element_type=jnp.float32) + bh[...]


def _encoder(x, p):
    out_shapes = (
        jax.ShapeDtypeStruct((N, 12), jnp.float32),
        jax.ShapeDtypeStruct((N, 8), jnp.float32),
        jax.ShapeDtypeStruct((N, 22), jnp.float32),
    )
    return pl.pallas_call(
        _encoder_body,
        out_shape=out_shapes,
    )(x, p['nn1_w1'], p['nn1_b1'].reshape(1, -1), p['nn1_w2'], p['nn1_b2'].reshape(1, -1),
      p['nn1_w3'], p['nn1_b3'].reshape(1, -1), p['c1_ws'], p['c1_bs'].reshape(1, -1),
      p['c1_wh'], p['c1_bh'].reshape(1, -1))


def _heads_body(xc_ref, aggr2_ref, x_ref,
                wrel, brel, wroot,
                n2w1, n2b1, n2w2, n2b2, n2w3, n2b3, n2w4, n2b4,
                n3w1, n3b1, n3w2, n3b2, n3w3, n3b3, n3w4, n3b4,
                ids_ref, p4_ref):
    xc = xc_ref[...]
    xg = (jnp.dot(aggr2_ref[...], wrel[...], preferred_element_type=jnp.float32)
          + brel[...]
          + jnp.dot(xc, wroot[...], preferred_element_type=jnp.float32))
    h2 = _leaky(jnp.dot(xg, n2w1[...], preferred_element_type=jnp.float32) + n2b1[...])
    h2 = _leaky(jnp.dot(h2, n2w2[...], preferred_element_type=jnp.float32) + n2b2[...])
    h2 = _leaky(jnp.dot(h2, n2w3[...], preferred_element_type=jnp.float32) + n2b3[...])
    cand_ids = _leaky(jnp.dot(h2, n2w4[...], preferred_element_type=jnp.float32) + n2b4[...])
    ids_ref[...] = cand_ids
    inp3 = jnp.concatenate([xg, cand_ids, x_ref[...]], axis=-1)
    h3 = _leaky(jnp.dot(inp3, n3w1[...], preferred_element_type=jnp.float32) + n3b1[...])
    h3 = _leaky(jnp.dot(h3, n3w2[...], preferred_element_type=jnp.float32) + n3b2[...])
    h3 = _leaky(jnp.dot(h3, n3w3[...], preferred_element_type=jnp.float32) + n3b3[...])
    p4_ref[...] = _leaky(jnp.dot(h3, n3w4[...], preferred_element_type=jnp.float32) + n3b4[...])


def _heads(xc, aggr2, x, p):
    out_shapes = (
        jax.ShapeDtypeStruct((N, 6), jnp.float32),
        jax.ShapeDtypeStruct((N, 6), jnp.float32),
    )
    return pl.pallas_call(
        _heads_body,
        out_shape=out_shapes,
    )(xc, aggr2, x,
      p['c2_wrel'], p['c2_brel'].reshape(1, -1), p['c2_wroot'],
      p['nn2_w1'], p['nn2_b1'].reshape(1, -1), p['nn2_w2'], p['nn2_b2'].reshape(1, -1),
      p['nn2_w3'], p['nn2_b3'].reshape(1, -1), p['nn2_w4'], p['nn2_b4'].reshape(1, -1),
      p['nn3_w1'], p['nn3_b1'].reshape(1, -1), p['nn3_w2'], p['nn3_b2'].reshape(1, -1),
      p['nn3_w3'], p['nn3_b3'].reshape(1, -1), p['nn3_w4'], p['nn3_b4'].reshape(1, -1))


def _knn(s, k, chunk=2000):
    n = s.shape[0]
    sq = jnp.sum(s * s, axis=-1)
    outs = []
    for st in range(0, n, chunk):
        q = s[st:st + chunk]
        d2 = jnp.sum(q * q, axis=-1, keepdims=True) - 2.0 * (q @ s.T) + sq[None, :]
        _, idx = jax.lax.top_k(-d2, k)
        outs.append(idx)
    return jnp.concatenate(outs, axis=0)


def kernel(x, ygen_id, ygen, ycand_id, ycand, params):
    p = params
    x1, s, hp = _encoder(x, p)
    idx = _knn(jax.lax.stop_gradient(s), K)
    src = idx.reshape(-1)
    dst = jnp.repeat(jnp.arange(N), K)
    d2 = jnp.sum((s[dst] - s[src]) ** 2, axis=-1)
    ew = jnp.exp(-10.0 * d2)
    msg = hp[src] * ew[:, None]
    agg_mean = jax.ops.segment_sum(msg, dst, num_segments=N) / float(K)
    agg_max = jax.ops.segment_max(msg, dst, num_segments=N)
    xc = jnp.concatenate([x1, agg_mean, agg_max], axis=-1) @ p['c1_wo'] + p['c1_bo']
    msg2 = xc[src] * ew[:, None]
    aggr2 = jax.ops.segment_sum(msg2, dst, num_segments=N)
    cand_ids, cand_p4 = _heads(xc, aggr2, x, p)
    return (cand_ids, cand_p4, ygen_id, ygen, ycand_id, ycand)


# speedup vs baseline: 5.8084x; 5.8084x over previous
import functools

import jax
import jax.numpy as jnp
from jax import lax
from jax.experimental import pallas as pl
from jax.experimental.pallas import tpu as pltpu
from jax.experimental.pallas import tpu_sc as plsc

N = 10000
K = 40
NP = 10240
NW = 32
RPW = NP // NW
GRP = 16
NGRP = RPW // GRP
CHUNK = 512
NCH = NP // CHUNK
BUFCAP = 192
DBLK = 256
BIG = 3.0e38


def _leaky(v):
    return jnp.where(v >= 0, v, 0.5 * v)


def _encoder_body(x_ref, w1, b1, w2, b2, w3, b3, ws, bs, wh, bh,
                  x1_ref, s_ref, hp_ref):
    x = x_ref[...]
    h = _leaky(jnp.dot(x, w1[...], preferred_element_type=jnp.float32) + b1[...])
    h = _leaky(jnp.dot(h, w2[...], preferred_element_type=jnp.float32) + b2[...])
    x1 = _leaky(jnp.dot(h, w3[...], preferred_element_type=jnp.float32) + b3[...])
    x1_ref[...] = x1
    s_ref[...] = jnp.dot(x1, ws[...], preferred_element_type=jnp.float32) + bs[...]
    hp_ref[...] = jnp.dot(x1, wh[...], preferred_element_type=jnp.float32) + bh[...]


def _encoder(x, p):
    out_shapes = (
        jax.ShapeDtypeStruct((N, 12), jnp.float32),
        jax.ShapeDtypeStruct((N, 8), jnp.float32),
        jax.ShapeDtypeStruct((N, 22), jnp.float32),
    )
    return pl.pallas_call(_encoder_body, out_shape=out_shapes)(
        x, p['nn1_w1'], p['nn1_b1'].reshape(1, -1), p['nn1_w2'], p['nn1_b2'].reshape(1, -1),
        p['nn1_w3'], p['nn1_b3'].reshape(1, -1), p['c1_ws'], p['c1_bs'].reshape(1, -1),
        p['c1_wh'], p['c1_bh'].reshape(1, -1))


def _d2_body(s_ref, q_ref, d2_ref, hi_ref):
    q = q_ref[...]
    s = s_ref[...]
    qq = jnp.sum(q * q, axis=1, keepdims=True)
    sq = jnp.sum(s * s, axis=1).reshape(1, NP)
    prod = lax.dot_general(q, s, (((1,), (1,)), ((), ())),
                           preferred_element_type=jnp.float32)
    d2 = jnp.maximum(qq - 2.0 * prod + sq, 0.0)
    d2_ref[...] = d2
    m = d2[:, 0:128]
    for a in range(1, NP // 128):
        m = jnp.minimum(m, d2[:, a * 128:(a + 1) * 128])
    mb = lax.bitcast_convert_type(m, jnp.int32)
    t = jnp.zeros((DBLK, 1), jnp.int32)
    for b in range(30, -1, -1):
        cand = t + (1 << b)
        cnt = jnp.sum((mb < cand).astype(jnp.float32), axis=1, keepdims=True)
        t = jnp.where(cnt <= 39.0, cand, t)
    hi_ref[...] = lax.bitcast_convert_type(t + 1, jnp.float32)


def _d2_thresh(s_pad):
    grid = NP // DBLK
    d2, hi = pl.pallas_call(
        _d2_body,
        grid=(grid,),
        in_specs=[
            pl.BlockSpec((NP, 8), lambda i: (0, 0)),
            pl.BlockSpec((DBLK, 8), lambda i: (i, 0)),
        ],
        out_specs=[
            pl.BlockSpec((DBLK, NP), lambda i: (i, 0)),
            pl.BlockSpec((DBLK, 1), lambda i: (i, 0)),
        ],
        out_shape=(
            jax.ShapeDtypeStruct((NP, NP), jnp.float32),
            jax.ShapeDtypeStruct((NP, 1), jnp.float32),
        ),
    )(s_pad, s_pad)
    return d2.reshape(NP * NP), hi.reshape(NP)


def _sc_select_body(d2_hbm, hi_hbm, idx_hbm, ew_hbm,
                    stage, vbuf, ibuf, thrv, oidx, oval, oew, sem):
    wid = lax.axis_index("s") * 2 + lax.axis_index("c")
    iota = lax.iota(jnp.int32, 16)
    zero16 = jnp.zeros((16,), jnp.int32)
    iota_c = iota * CHUNK

    def group_body(g, carry):
        r0 = wid * RPW + g * GRP
        pltpu.sync_copy(hi_hbm.at[pl.ds(r0, 16)], thrv)
        thr = thrv[...]

        def init_body(k2, c2):
            vbuf[pl.ds(k2 * 16, 16)] = jnp.full((16,), BIG, jnp.float32)
            return c2
        lax.fori_loop(0, BUFCAP, init_body, 0)

        def chunk_body(c, offs):
            copies = []
            for l in range(16):
                cp = pltpu.async_copy(
                    d2_hbm.at[pl.ds((r0 + l) * NP + c * CHUNK, CHUNK)],
                    stage.at[pl.ds(l * CHUNK, CHUNK)], sem)
                copies.append(cp)
            for cp in copies:
                cp.wait()
            jbase = c * CHUNK

            def j_body(j8, offs2):
                for u in range(8):
                    jloc = j8 * 8 + u
                    col = plsc.load_gather(stage, [iota_c + jloc])
                    msk = (col < thr) & (offs2 < BUFCAP * 16)
                    plsc.store_scatter(vbuf, [offs2 + iota], col, mask=msk)
                    plsc.store_scatter(ibuf, [offs2 + iota],
                                       jnp.full((16,), jbase + jloc, jnp.int32), mask=msk)
                    offs2 = offs2 + jnp.where(msk, 16, 0)
                return offs2
            return lax.fori_loop(0, CHUNK // 8, j_body, offs)

        offs = lax.fori_loop(0, NCH, chunk_body, zero16)
        nslot = jnp.max(offs) // 16

        def bis_body(tstep, tval):
            cand = tval + (1 << (30 - tstep))

            def cnt_body(k2, cnt2):
                bits = plsc.bitcast(vbuf[pl.ds(k2 * 16, 16)], jnp.int32)
                return cnt2 + jnp.where(bits < cand, 1, 0)
            cnt = lax.fori_loop(0, nslot, cnt_body, zero16)
            return jnp.where(cnt <= 39, cand, tval)

        tstar = lax.fori_loop(0, 31, bis_body, zero16)

        def cl_body(k2, cnt2):
            bits = plsc.bitcast(vbuf[pl.ds(k2 * 16, 16)], jnp.int32)
            return cnt2 + jnp.where(bits < tstar, 1, 0)
        c_less = lax.fori_loop(0, nslot, cl_body, zero16)
        need = 40 - c_less

        def zero_out(k2, c2):
            oidx[pl.ds(k2 * 16, 16)] = zero16
            oval[pl.ds(k2 * 16, 16)] = jnp.zeros((16,), jnp.float32)
            return c2
        lax.fori_loop(0, K, zero_out, 0)

        def sel_body(k2, carry2):
            kpos, eqtot = carry2
            v = vbuf[pl.ds(k2 * 16, 16)]
            bits = plsc.bitcast(v, jnp.int32)
            less = bits < tstar
            eq = bits == tstar
            keep = (less | (eq & (eqtot < need))) & (kpos < 640)
            plsc.store_scatter(oidx, [kpos], ibuf[pl.ds(k2 * 16, 16)], mask=keep)
            plsc.store_scatter(oval, [kpos], v, mask=keep)
            kpos = kpos + jnp.where(keep, 16, 0)
            eqtot = eqtot + jnp.where(eq, 1, 0)
            return (kpos, eqtot)

        lax.fori_loop(0, nslot, sel_body, (iota, zero16))

        for k in range(K):
            oew[pl.ds(k * 16, 16)] = jnp.exp(-10.0 * oval[pl.ds(k * 16, 16)])

        gg = wid * NGRP + g
        pltpu.sync_copy(oidx, idx_hbm.at[pl.ds(gg * 640, 640)])
        pltpu.sync_copy(oew, ew_hbm.at[pl.ds(gg * 640, 640)])
        return carry

    lax.fori_loop(0, NGRP, group_body, 0)


def _sc_select(d2_flat, hi):
    mesh = plsc.VectorSubcoreMesh(core_axis_name="c", subcore_axis_name="s")
    f = pl.kernel(
        _sc_select_body,
        out_type=(
            jax.ShapeDtypeStruct((NP * K,), jnp.int32),
            jax.ShapeDtypeStruct((NP * K,), jnp.float32),
        ),
        mesh=mesh,
        compiler_params=pltpu.CompilerParams(needs_layout_passes=False, use_tc_tiling_on_sc=False),
        scratch_types=[
            pltpu.VMEM((16 * CHUNK,), jnp.float32),
            pltpu.VMEM((BUFCAP * 16,), jnp.float32),
            pltpu.VMEM((BUFCAP * 16,), jnp.int32),
            pltpu.VMEM((16,), jnp.float32),
            pltpu.VMEM((GRP * K,), jnp.int32),
            pltpu.VMEM((GRP * K,), jnp.float32),
            pltpu.VMEM((GRP * K,), jnp.float32),
            pltpu.SemaphoreType.DMA,
        ],
    )
    return f(d2_flat, hi)


def _sc_agg_hp_body(hp_hbm, idx_hbm, ew_hbm, agg_hbm,
                    idxv, ewv, grows, oagg, sem):
    wid = lax.axis_index("s") * 2 + lax.axis_index("c")

    def group_body(g, carry):
        gg = wid * NGRP + g
        r0 = wid * RPW + g * GRP
        pltpu.sync_copy(idx_hbm.at[pl.ds(gg * 640, 640)], idxv)
        pltpu.sync_copy(ew_hbm.at[pl.ds(gg * 640, 640)], ewv)
        copies = []
        for u in range(640 // 128):
            cp = pltpu.async_copy(hp_hbm.at[idxv.at[pl.ds(u * 128, 128)]],
                                  grows.at[pl.ds(u * 128, 128)], sem)
            copies.append(cp)
        for cp in copies:
            cp.wait()

        def row_body(r, carry2):
            acc0 = jnp.zeros((16,), jnp.float32)
            acc1 = jnp.zeros((16,), jnp.float32)
            mx0 = jnp.full((16,), -BIG, jnp.float32)
            mx1 = jnp.full((16,), -BIG, jnp.float32)
            for k in range(K):
                pos = k * 16 + r
                w = plsc.load_gather(ewv, [jnp.full((16,), pos, jnp.int32)])
                g0 = grows[pos, pl.ds(0, 16)] * w
                g1 = grows[pos, pl.ds(16, 16)] * w
                acc0 = acc0 + g0
                acc1 = acc1 + g1
                mx0 = jnp.maximum(mx0, g0)
                mx1 = jnp.maximum(mx1, g1)
            oagg[r, pl.ds(0, 16)] = acc0 * jnp.float32(1.0 / K)
            oagg[r, pl.ds(16, 16)] = acc1 * jnp.float32(1.0 / K)
            oagg[r, pl.ds(32, 16)] = mx0
            oagg[r, pl.ds(48, 16)] = mx1
            return carry2

        lax.fori_loop(0, GRP, row_body, 0)
        pltpu.sync_copy(oagg, agg_hbm.at[pl.ds(r0, 16)])
        return carry

    lax.fori_loop(0, NGRP, group_body, 0)


def _sc_agg_hp(hp_pad, idx_flat, ew_flat):
    mesh = plsc.VectorSubcoreMesh(core_axis_name="c", subcore_axis_name="s")
    f = pl.kernel(
        _sc_agg_hp_body,
        out_type=jax.ShapeDtypeStruct((NP, 64), jnp.float32),
        mesh=mesh,
        compiler_params=pltpu.CompilerParams(needs_layout_passes=False, use_tc_tiling_on_sc=False),
        scratch_types=[
            pltpu.VMEM((GRP * K,), jnp.int32),
            pltpu.VMEM((GRP * K,), jnp.float32),
            pltpu.VMEM((GRP * K, 32), jnp.float32),
            pltpu.VMEM((GRP, 64), jnp.float32),
            pltpu.SemaphoreType.DMA,
        ],
    )
    return f(hp_pad, idx_flat, ew_flat)


def _sc_agg_xc_body(xc_hbm, idx_hbm, ew_hbm, agg_hbm,
                    idxv, ewv, grows, oagg, sem):
    wid = lax.axis_index("s") * 2 + lax.axis_index("c")

    def group_body(g, carry):
        gg = wid * NGRP + g
        r0 = wid * RPW + g * GRP
        pltpu.sync_copy(idx_hbm.at[pl.ds(gg * 640, 640)], idxv)
        pltpu.sync_copy(ew_hbm.at[pl.ds(gg * 640, 640)], ewv)
        copies = []
        for u in range(640 // 128):
            cp = pltpu.async_copy(xc_hbm.at[idxv.at[pl.ds(u * 128, 128)]],
                                  grows.at[pl.ds(u * 128, 128)], sem)
            copies.append(cp)
        for cp in copies:
            cp.wait()

        def row_body(r, carry2):
            acc = [jnp.zeros((16,), jnp.float32) for _ in range(4)]
            for k in range(K):
                pos = k * 16 + r
                w = plsc.load_gather(ewv, [jnp.full((16,), pos, jnp.int32)])
                for q in range(4):
                    acc[q] = acc[q] + grows[pos, pl.ds(q * 16, 16)] * w
            for q in range(4):
                oagg[r, pl.ds(q * 16, 16)] = acc[q]
            return carry2

        lax.fori_loop(0, GRP, row_body, 0)
        pltpu.sync_copy(oagg, agg_hbm.at[pl.ds(r0, 16)])
        return carry

    lax.fori_loop(0, NGRP, group_body, 0)


def _sc_agg_xc(xc_pad, idx_flat, ew_flat):
    mesh = plsc.VectorSubcoreMesh(core_axis_name="c", subcore_axis_name="s")
    f = pl.kernel(
        _sc_agg_xc_body,
        out_type=jax.ShapeDtypeStruct((NP, 64), jnp.float32),
        mesh=mesh,
        compiler_params=pltpu.CompilerParams(needs_layout_passes=False, use_tc_tiling_on_sc=False),
        scratch_types=[
            pltpu.VMEM((GRP * K,), jnp.int32),
            pltpu.VMEM((GRP * K,), jnp.float32),
            pltpu.VMEM((GRP * K, 64), jnp.float32),
            pltpu.VMEM((GRP, 64), jnp.float32),
            pltpu.SemaphoreType.DMA,
        ],
    )
    return f(xc_pad, idx_flat, ew_flat)


def _xc_body(x1_ref, agg_ref, w1, w2, w3, bo, xc_ref):
    agg = agg_ref[...]
    mean = agg[:, 0:22]
    mx = agg[:, 32:54]
    xc_ref[...] = (jnp.dot(x1_ref[...], w1[...], preferred_element_type=jnp.float32)
                   + jnp.dot(mean, w2[...], preferred_element_type=jnp.float32)
                   + jnp.dot(mx, w3[...], preferred_element_type=jnp.float32)
                   + bo[...])


def _xc(x1_pad, agg, p):
    wo = p['c1_wo']
    return pl.pallas_call(
        _xc_body,
        out_shape=jax.ShapeDtypeStruct((NP, 64), jnp.float32),
    )(x1_pad, agg, wo[0:12], wo[12:34], wo[34:56], p['c1_bo'].reshape(1, -1))


def _heads_body(xc_ref, aggr2_ref, x_ref,
                wrel, brel, wroot,
                n2w1, n2b1, n2w2, n2b2, n2w3, n2b3, n2w4, n2b4,
                n3w1, n3b1, n3w2, n3b2, n3w3, n3b3, n3w4, n3b4,
                ids_ref, p4_ref):
    xc = xc_ref[...]
    xg = (jnp.dot(aggr2_ref[...], wrel[...], preferred_element_type=jnp.float32)
          + brel[...]
          + jnp.dot(xc, wroot[...], preferred_element_type=jnp.float32))
    h2 = _leaky(jnp.dot(xg, n2w1[...], preferred_element_type=jnp.float32) + n2b1[...])
    h2 = _leaky(jnp.dot(h2, n2w2[...], preferred_element_type=jnp.float32) + n2b2[...])
    h2 = _leaky(jnp.dot(h2, n2w3[...], preferred_element_type=jnp.float32) + n2b3[...])
    cand_ids = _leaky(jnp.dot(h2, n2w4[...], preferred_element_type=jnp.float32) + n2b4[...])
    ids_ref[...] = cand_ids
    inp3 = jnp.concatenate([xg, cand_ids, x_ref[...]], axis=-1)
    h3 = _leaky(jnp.dot(inp3, n3w1[...], preferred_element_type=jnp.float32) + n3b1[...])
    h3 = _leaky(jnp.dot(h3, n3w2[...], preferred_element_type=jnp.float32) + n3b2[...])
    h3 = _leaky(jnp.dot(h3, n3w3[...], preferred_element_type=jnp.float32) + n3b3[...])
    p4_ref[...] = _leaky(jnp.dot(h3, n3w4[...], preferred_element_type=jnp.float32) + n3b4[...])


def _heads(xc, aggr2, x, p):
    out_shapes = (
        jax.ShapeDtypeStruct((N, 6), jnp.float32),
        jax.ShapeDtypeStruct((N, 6), jnp.float32),
    )
    return pl.pallas_call(_heads_body, out_shape=out_shapes)(
        xc, aggr2, x,
        p['c2_wrel'], p['c2_brel'].reshape(1, -1), p['c2_wroot'],
        p['nn2_w1'], p['nn2_b1'].reshape(1, -1), p['nn2_w2'], p['nn2_b2'].reshape(1, -1),
        p['nn2_w3'], p['nn2_b3'].reshape(1, -1), p['nn2_w4'], p['nn2_b4'].reshape(1, -1),
        p['nn3_w1'], p['nn3_b1'].reshape(1, -1), p['nn3_w2'], p['nn3_b2'].reshape(1, -1),
        p['nn3_w3'], p['nn3_b3'].reshape(1, -1), p['nn3_w4'], p['nn3_b4'].reshape(1, -1))


def kernel(x, ygen_id, ygen, ycand_id, ycand, params):
    p = params
    x1, s, hp = _encoder(x, p)
    s_pad = jnp.concatenate(
        [s, jnp.full((NP - N, 8), 1.0e15, jnp.float32)], axis=0)
    d2_flat, hi = _d2_thresh(s_pad)
    idx_flat, ew_flat = _sc_select(d2_flat, hi)
    hp_pad = jnp.pad(hp, ((0, NP - N), (0, 10)))
    agg = _sc_agg_hp(hp_pad, idx_flat, ew_flat)
    x1_pad = jnp.pad(x1, ((0, NP - N), (0, 0)))
    xc_pad = _xc(x1_pad, agg, p)
    aggr2 = _sc_agg_xc(xc_pad, idx_flat, ew_flat)
    cand_ids, cand_p4 = _heads(xc_pad[:N], aggr2[:N], x, p)
    return (cand_ids, cand_p4, ygen_id, ygen, ycand_id, ycand)
